# Initial kernel scaffold; baseline (speedup 1.0000x reference)
import jax
import jax.numpy as jnp
from jax.experimental import pallas as pl


def kernel(xyz1, xyz2):
    raise NotImplementedError("write your pallas kernel here")


# speedup vs baseline: 1.9340x; 1.9340x over previous
import functools

import jax
import jax.numpy as jnp
from jax import lax
from jax.experimental import pallas as pl
from jax.experimental.pallas import tpu as pltpu


def _chamfer_tile_kernel(x1_ref, x2t_ref, d1_ref, i1_ref, d2_ref, i2_ref,
                         *, tn, n, m):
    ni = pl.program_id(1)
    x1 = x1_ref[0]
    x2t = x2t_ref[0]

    inner = jnp.dot(x1, x2t, preferred_element_type=jnp.float32)
    xn = jnp.sum(x1 * x1, axis=1, keepdims=True)
    yn = jnp.sum(x2t * x2t, axis=0, keepdims=True)
    d = jnp.maximum(xn + yn - 2.0 * inner, 0.0)

    dmin = jnp.min(d, axis=1, keepdims=True)
    iota_m = lax.broadcasted_iota(jnp.int32, (tn, m), 1)
    imin = jnp.min(jnp.where(d == dmin, iota_m, m), axis=1,
                   keepdims=True)
    d1_ref[0] = dmin
    i1_ref[0] = imin

    cmin = jnp.min(d, axis=0, keepdims=True)
    iota_n = lax.broadcasted_iota(jnp.int32, (tn, m), 0)
    cidx = jnp.min(jnp.where(d == cmin, iota_n, n), axis=0,
                   keepdims=True) + ni * tn

    @pl.when(ni == 0)
    def _init():
        d2_ref[0] = cmin
        i2_ref[0] = cidx

    @pl.when(ni != 0)
    def _merge():
        prev_d = d2_ref[0]
        prev_i = i2_ref[0]
        better = cmin < prev_d
        d2_ref[0] = jnp.where(better, cmin, prev_d)
        i2_ref[0] = jnp.where(better, cidx, prev_i)


def _chamfer(xyz1, xyz2, tn):
    b, n, _ = xyz1.shape
    m = xyz2.shape[1]
    f32 = jnp.float32
    i32 = jnp.int32

    pad = jnp.zeros((b, n, 5), f32)
    x1p = jnp.concatenate([xyz1, pad], axis=-1)
    x2t = jnp.concatenate([jnp.swapaxes(xyz2, 1, 2),
                           jnp.zeros((b, 5, m), f32)], axis=1)

    grid = (b, n // tn)
    d1, i1, d2, i2 = pl.pallas_call(
        functools.partial(_chamfer_tile_kernel, tn=tn, n=n, m=m),
        grid=grid,
        in_specs=[
            pl.BlockSpec((1, tn, 8), lambda bi, ni: (bi, ni, 0)),
            pl.BlockSpec((1, 8, m), lambda bi, ni: (bi, 0, 0)),
        ],
        out_specs=[
            pl.BlockSpec((1, tn, 1), lambda bi, ni: (bi, ni, 0)),
            pl.BlockSpec((1, tn, 1), lambda bi, ni: (bi, ni, 0)),
            pl.BlockSpec((1, 1, m), lambda bi, ni: (bi, 0, 0)),
            pl.BlockSpec((1, 1, m), lambda bi, ni: (bi, 0, 0)),
        ],
        out_shape=[
            jax.ShapeDtypeStruct((b, n, 1), f32),
            jax.ShapeDtypeStruct((b, n, 1), i32),
            jax.ShapeDtypeStruct((b, 1, m), f32),
            jax.ShapeDtypeStruct((b, 1, m), i32),
        ],
        compiler_params=pltpu.CompilerParams(
            dimension_semantics=("parallel", "arbitrary"),
        ),
    )(x1p, x2t)

    return (d1.reshape(b, n), d2.reshape(b, m),
            i1.reshape(b, n), i2.reshape(b, m))


def kernel(xyz1, xyz2):
    return _chamfer(xyz1, xyz2, tn=256)


# speedup vs baseline: 2.5190x; 1.3025x over previous
import functools

import jax
import jax.numpy as jnp
from jax import lax
from jax.experimental import pallas as pl
from jax.experimental.pallas import tpu as pltpu


def _chamfer_tile_kernel(x1_ref, x2ts_ref, d1_ref, i1_ref, d2_ref, i2_ref,
                         *, tn, n, m):
    ni = pl.program_id(1)
    x1 = x1_ref[0]
    x2ts = x2ts_ref[0]

    inner2 = jnp.dot(x1, x2ts, preferred_element_type=jnp.float32)
    xn = jnp.sum(x1 * x1, axis=1, keepdims=True)
    yn = jnp.sum(x2ts * x2ts, axis=0, keepdims=True) * 0.25
    d = (xn + yn) + inner2

    def _tree_min(parts):
        while len(parts) > 1:
            nxt = [jnp.minimum(parts[k], parts[k + 1])
                   for k in range(0, len(parts) - 1, 2)]
            if len(parts) % 2:
                nxt.append(parts[-1])
            parts = nxt
        return parts[0]

    nc = m // 128
    cs = [d[:, j * 128:(j + 1) * 128] for j in range(nc)]
    run = _tree_min(cs)
    idxc = jnp.zeros((tn, 128), jnp.int32)
    for j in range(nc - 1, -1, -1):
        idxc = jnp.where(cs[j] == run, j, idxc)
    rmin = jnp.min(run, axis=1, keepdims=True)
    lane = lax.broadcasted_iota(jnp.int32, (tn, 128), 1)
    cand = idxc * 128 + lane
    imin = jnp.min(jnp.where(run == rmin, cand, m), axis=1,
                   keepdims=True)
    d1_ref[0] = jnp.maximum(rmin, 0.0)
    i1_ref[0] = imin

    nr = tn // 8
    rows = [d[i * 8:(i + 1) * 8, :] for i in range(nr)]
    run2 = _tree_min(rows)
    idxr = jnp.zeros((8, m), jnp.int32)
    for i in range(nr - 1, -1, -1):
        idxr = jnp.where(rows[i] == run2, i, idxr)
    rawc = jnp.min(run2, axis=0, keepdims=True)
    cmin = jnp.maximum(rawc, 0.0)
    sub = lax.broadcasted_iota(jnp.int32, (8, m), 0)
    cand2 = idxr * 8 + sub
    cidx = jnp.min(jnp.where(run2 == rawc, cand2, n), axis=0,
                   keepdims=True) + ni * tn

    @pl.when(ni == 0)
    def _init():
        d2_ref[0] = cmin
        i2_ref[0] = cidx

    @pl.when(ni != 0)
    def _merge():
        prev_d = d2_ref[0]
        prev_i = i2_ref[0]
        better = cmin < prev_d
        d2_ref[0] = jnp.where(better, cmin, prev_d)
        i2_ref[0] = jnp.where(better, cidx, prev_i)


def _chamfer(xyz1, xyz2, tn):
    b, n, _ = xyz1.shape
    m = xyz2.shape[1]
    f32 = jnp.float32
    i32 = jnp.int32

    pad = jnp.zeros((b, n, 5), f32)
    x1p = jnp.concatenate([xyz1, pad], axis=-1)
    x2ts = jnp.concatenate([jnp.swapaxes(xyz2, 1, 2) * -2.0,
                            jnp.zeros((b, 5, m), f32)], axis=1)

    grid = (b, n // tn)
    d1, i1, d2, i2 = pl.pallas_call(
        functools.partial(_chamfer_tile_kernel, tn=tn, n=n, m=m),
        grid=grid,
        in_specs=[
            pl.BlockSpec((1, tn, 8), lambda bi, ni: (bi, ni, 0)),
            pl.BlockSpec((1, 8, m), lambda bi, ni: (bi, 0, 0)),
        ],
        out_specs=[
            pl.BlockSpec((1, tn, 1), lambda bi, ni: (bi, ni, 0)),
            pl.BlockSpec((1, tn, 1), lambda bi, ni: (bi, ni, 0)),
            pl.BlockSpec((1, 1, m), lambda bi, ni: (bi, 0, 0)),
            pl.BlockSpec((1, 1, m), lambda bi, ni: (bi, 0, 0)),
        ],
        out_shape=[
            jax.ShapeDtypeStruct((b, n, 1), f32),
            jax.ShapeDtypeStruct((b, n, 1), i32),
            jax.ShapeDtypeStruct((b, 1, m), f32),
            jax.ShapeDtypeStruct((b, 1, m), i32),
        ],
        compiler_params=pltpu.CompilerParams(
            dimension_semantics=("parallel", "arbitrary"),
        ),
    )(x1p, x2ts)

    return (d1.reshape(b, n), d2.reshape(b, m),
            i1.reshape(b, n), i2.reshape(b, m))


def kernel(xyz1, xyz2):
    return _chamfer(xyz1, xyz2, tn=256)
